# baseline (speedup 1.0000x reference)
import functools
import math

import jax
import jax.numpy as jnp
from jax import lax
from jax.experimental import pallas as pl
from jax.experimental.pallas import tpu as pltpu
from jax.experimental.pallas import tpu_sc as plsc

N_NODES = 10000
N_EDGES = 320000
IN_C = 128
HID = 64
PAD_W = 128
ALPHA = 0.1
THETA = 0.5

NC, NS = 2, 16
NW = NC * NS
CH = 80
NCH = N_EDGES // (NW * CH)
VROWS = 2 * N_NODES
ROWS_PER_TILE = VROWS // NS

_sc_mesh = plsc.VectorSubcoreMesh(core_axis_name="c", subcore_axis_name="s")


def _zero_acc_slice(buf, acc_sh, base):
    z16 = jnp.zeros((16,), jnp.float32)
    width = buf.shape[1]

    def zrow(i, carry):
        for j in range(width // 16):
            buf[i, pl.ds(j * 16, 16)] = z16
        return carry

    lax.fori_loop(0, CH, zrow, 0)
    for k in range(ROWS_PER_TILE // CH):
        pltpu.sync_copy(buf, acc_sh.at[pl.ds(base + k * CH, CH)])
    rem = ROWS_PER_TILE % CH
    if rem:
        pltpu.sync_copy(buf.at[pl.ds(0, rem)],
                        acc_sh.at[pl.ds(base + (ROWS_PER_TILE // CH) * CH, rem)])


def _spmm_body(g_hbm, row_hbm, col_hbm, out_hbm,
               idx_r, idx_c, b0, b1, b2, b3, acc_sh,
               g0, g1, g2, g3, s0, s1, s2, s3):
    bufs = (b0, b1, b2, b3)
    gsem = (g0, g1, g2, g3)
    ssem = (s0, s1, s2, s3)
    c = lax.axis_index("c")
    s = lax.axis_index("s")
    w = c * NS + s
    pltpu.sync_copy(row_hbm.at[w], idx_r)
    pltpu.sync_copy(col_hbm.at[w], idx_c)
    base = s * ROWS_PER_TILE

    @pl.when(c == 0)
    def _():
        pltpu.sync_copy(g_hbm.at[pl.ds(base, ROWS_PER_TILE)],
                        acc_sh.at[pl.ds(base, ROWS_PER_TILE)])

    @pl.when(c == 1)
    def _():
        _zero_acc_slice(bufs[0], acc_sh, base)

    plsc.subcore_barrier()

    def _gather(j, b):
        pltpu.async_copy(g_hbm.at[idx_r.at[j]], bufs[b], gsem[b])

    def _wait_gather(j, b):
        pltpu.make_async_copy(g_hbm.at[idx_r.at[j]], bufs[b], gsem[b]).wait()

    def _scatter(j, b):
        pltpu.async_copy(bufs[b], acc_sh.at[idx_c.at[j]], ssem[b], add=True)

    def _wait_scatter(b):
        pltpu.make_async_copy(bufs[b], acc_sh.at[idx_c.at[0]], ssem[b]).wait()

    _gather(0, 0)
    _gather(1, 1)

    def group(t, carry):
        j0 = 4 * t
        for b in range(4):
            j = j0 + b
            sb = (b + 2) % 4

            @pl.when(j >= 2)
            def _():
                _wait_scatter(sb)

            @pl.when(j + 2 < NCH)
            def _():
                _gather(j + 2, sb)

            _wait_gather(j, b)
            _scatter(j, b)
        return carry

    lax.fori_loop(0, NCH // 4, group, 0)
    _wait_gather(NCH - 1, 0)
    _scatter(NCH - 1, 0)
    for b in (0, 2, 3):
        _wait_scatter(b)

    plsc.subcore_barrier()
    pltpu.sync_copy(acc_sh.at[pl.ds(base, ROWS_PER_TILE)],
                    out_hbm.at[c].at[pl.ds(base, ROWS_PER_TILE)])


_spmm = pl.kernel(
    _spmm_body,
    out_type=jax.ShapeDtypeStruct((NC, VROWS, HID), jnp.float32),
    mesh=_sc_mesh,
    compiler_params=pltpu.CompilerParams(use_tc_tiling_on_sc=False),
    scratch_types=[
        pltpu.VMEM((NCH, CH), jnp.int32),
        pltpu.VMEM((NCH, CH), jnp.int32),
        pltpu.VMEM((CH, HID), jnp.float32),
        pltpu.VMEM((CH, HID), jnp.float32),
        pltpu.VMEM((CH, HID), jnp.float32),
        pltpu.VMEM((CH, HID), jnp.float32),
        pltpu.VMEM_SHARED((VROWS, HID), jnp.float32),
        pltpu.SemaphoreType.DMA,
        pltpu.SemaphoreType.DMA,
        pltpu.SemaphoreType.DMA,
        pltpu.SemaphoreType.DMA,
        pltpu.SemaphoreType.DMA,
        pltpu.SemaphoreType.DMA,
        pltpu.SemaphoreType.DMA,
        pltpu.SemaphoreType.DMA,
    ],
)


def _deg_body(col_hbm, out_hbm, idx_c, buf, acc_sh):
    c = lax.axis_index("c")
    s = lax.axis_index("s")
    w = c * NS + s
    pltpu.sync_copy(col_hbm.at[w], idx_c)
    base = s * ROWS_PER_TILE
    _zero_acc_slice(buf, acc_sh, base)

    one16 = jnp.ones((16,), jnp.float32)

    def orow(i, carry):
        for j in range(HID // 16):
            buf[i, pl.ds(j * 16, 16)] = one16
        return carry

    lax.fori_loop(0, CH, orow, 0)
    plsc.subcore_barrier()

    def chunk(j, carry):
        pltpu.sync_copy(buf, acc_sh.at[idx_c.at[j]], add=True)
        return carry

    lax.fori_loop(0, NCH, chunk, 0)
    plsc.subcore_barrier()
    pltpu.sync_copy(acc_sh.at[pl.ds(base, ROWS_PER_TILE)],
                    out_hbm.at[c].at[pl.ds(base, ROWS_PER_TILE)])


_deg = pl.kernel(
    _deg_body,
    out_type=jax.ShapeDtypeStruct((NC, VROWS, HID), jnp.float32),
    mesh=_sc_mesh,
    compiler_params=pltpu.CompilerParams(use_tc_tiling_on_sc=False),
    scratch_types=[
        pltpu.VMEM((NCH, CH), jnp.int32),
        pltpu.VMEM((CH, HID), jnp.float32),
        pltpu.VMEM_SHARED((VROWS, HID), jnp.float32),
    ],
)


BR = 1000
GRID = N_NODES // BR


def _embed_body(x_ref, we_ref, b_ref, x0_ref):
    x0 = jnp.dot(x_ref[...], we_ref[...], preferred_element_type=jnp.float32)
    x0 = jnp.maximum(x0 + b_ref[...], 0.0)
    x0_ref[...] = jnp.concatenate(
        [x0, jnp.zeros((BR, PAD_W - HID), jnp.float32)], axis=1)


_embed = pl.pallas_call(
    _embed_body,
    grid=(GRID,),
    in_specs=[
        pl.BlockSpec((BR, IN_C), lambda i: (i, 0)),
        pl.BlockSpec((IN_C, HID), lambda i: (0, 0)),
        pl.BlockSpec((1, HID), lambda i: (0, 0)),
    ],
    out_specs=pl.BlockSpec((BR, PAD_W), lambda i: (i, 0)),
    out_shape=jax.ShapeDtypeStruct((N_NODES, PAD_W), jnp.float32),
)


def _scale_body(dp_ref, x0_ref, g_ref, v_ref, u_ref):
    deg = dp_ref[0] + dp_ref[1] + 1.0
    dv = lax.rsqrt(deg)
    x0 = x0_ref[...]
    g_ref[...] = dv * x0
    v_ref[...] = (1.0 - ALPHA) * dv
    u_ref[...] = ALPHA * x0


_scale = pl.pallas_call(
    _scale_body,
    grid=(GRID,),
    in_specs=[
        pl.BlockSpec((NC, BR, PAD_W), lambda i: (0, i, 0)),
        pl.BlockSpec((BR, PAD_W), lambda i: (i, 0)),
    ],
    out_specs=[
        pl.BlockSpec((BR, PAD_W), lambda i: (i, 0)),
        pl.BlockSpec((BR, PAD_W), lambda i: (i, 0)),
        pl.BlockSpec((BR, PAD_W), lambda i: (i, 0)),
    ],
    out_shape=[
        jax.ShapeDtypeStruct((N_NODES, PAD_W), jnp.float32),
        jax.ShapeDtypeStruct((N_NODES, PAD_W), jnp.float32),
        jax.ShapeDtypeStruct((N_NODES, PAD_W), jnp.float32),
    ],
)


def _layer_body(acc_ref, v_ref, u_ref, w_ref, o_ref, *, beta, final):
    t = v_ref[...] * (acc_ref[0] + acc_ref[1]) + u_ref[...]
    out = (1.0 - beta) * t + beta * jnp.dot(
        t, w_ref[...], preferred_element_type=jnp.float32)
    if final:
        o_ref[...] = out[:, :HID]
    else:
        o_ref[...] = (1.0 / (1.0 - ALPHA)) * v_ref[...] * jnp.maximum(out, 0.0)


def _make_layer(beta, final):
    w_out = HID if final else PAD_W
    return pl.pallas_call(
        functools.partial(_layer_body, beta=beta, final=final),
        grid=(GRID,),
        in_specs=[
            pl.BlockSpec((NC, BR, PAD_W), lambda i: (0, i, 0)),
            pl.BlockSpec((BR, PAD_W), lambda i: (i, 0)),
            pl.BlockSpec((BR, PAD_W), lambda i: (i, 0)),
            pl.BlockSpec((PAD_W, PAD_W), lambda i: (0, 0)),
        ],
        out_specs=pl.BlockSpec((BR, w_out), lambda i: (i, 0)),
        out_shape=jax.ShapeDtypeStruct((N_NODES, w_out), jnp.float32),
    )


_layers = [_make_layer(math.log(THETA / l + 1.0), final=(l == 4))
           for l in range(1, 5)]


def kernel(x, adj_t, W_embed, b_embed, W1, W2, W3, W4):
    adj2 = adj_t.astype(jnp.int32) * 2
    row3 = adj2[0].reshape(NW, NCH, CH)
    col3 = adj2[1].reshape(NW, NCH, CH)
    wpad = [jnp.pad(W, ((0, PAD_W - HID), (0, PAD_W - HID)))
            for W in (W1, W2, W3, W4)]

    degp = _deg(col3)
    x0 = _embed(x, W_embed, b_embed.reshape(1, HID))
    g, v, u = _scale(degp.reshape(NC, N_NODES, PAD_W), x0)
    for lyr, W2p in zip(_layers, wpad):
        acc = _spmm(g.reshape(VROWS, HID), row3, col3)
        g = lyr(acc.reshape(NC, N_NODES, PAD_W), v, u, W2p)
    return g


# speedup vs baseline: 1.4261x; 1.4261x over previous
import functools
import math

import jax
import jax.numpy as jnp
from jax import lax
from jax.experimental import pallas as pl
from jax.experimental.pallas import tpu as pltpu
from jax.experimental.pallas import tpu_sc as plsc

N_NODES = 10000
N_EDGES = 320000
IN_C = 128
HID = 64
ALPHA = 0.1
THETA = 0.5

NC, NS = 2, 16
NW = NC * NS
CH = 80
NCH = N_EDGES // (NW * CH)
ROWS_PER_TILE = N_NODES // NS
DEG_W = 16

PROWS = N_NODES // 2
PW = 2 * HID

_sc_mesh = plsc.VectorSubcoreMesh(core_axis_name="c", subcore_axis_name="s")


def _zero_acc_slice(buf, acc_sh, base):
    z16 = jnp.zeros((16,), jnp.float32)
    width = buf.shape[1]

    def zrow(i, carry):
        for j in range(width // 16):
            buf[i, pl.ds(j * 16, 16)] = z16
        return carry

    lax.fori_loop(0, CH, zrow, 0)
    for k in range(ROWS_PER_TILE // CH):
        pltpu.sync_copy(buf, acc_sh.at[pl.ds(base + k * CH, CH)])
    rem = ROWS_PER_TILE % CH
    if rem:
        pltpu.sync_copy(buf.at[pl.ds(0, rem)],
                        acc_sh.at[pl.ds(base + (ROWS_PER_TILE // CH) * CH, rem)])


def _spmm_body(g_hbm, row_hbm, col_hbm, out_hbm,
               idx_r, idx_c, b0, b1, b2, b3, acc_sh,
               g0, g1, g2, g3, s0, s1, s2, s3):
    bufs = (b0, b1, b2, b3)
    gsem = (g0, g1, g2, g3)
    ssem = (s0, s1, s2, s3)
    c = lax.axis_index("c")
    s = lax.axis_index("s")
    w = c * NS + s
    pltpu.sync_copy(row_hbm.at[w], idx_r)
    pltpu.sync_copy(col_hbm.at[w], idx_c)
    base = s * ROWS_PER_TILE

    @pl.when(c == 0)
    def _():
        pltpu.sync_copy(g_hbm.at[pl.ds(base, ROWS_PER_TILE)],
                        acc_sh.at[pl.ds(base, ROWS_PER_TILE)])

    @pl.when(c == 1)
    def _():
        _zero_acc_slice(bufs[0], acc_sh, base)

    plsc.subcore_barrier()

    def _gather(j, b):
        pltpu.async_copy(g_hbm.at[idx_r.at[j]], bufs[b], gsem[b])

    def _wait_gather(j, b):
        pltpu.make_async_copy(g_hbm.at[idx_r.at[j]], bufs[b], gsem[b]).wait()

    def _scatter(j, b):
        pltpu.async_copy(bufs[b], acc_sh.at[idx_c.at[j]], ssem[b], add=True)

    def _wait_scatter(b):
        pltpu.make_async_copy(bufs[b], acc_sh.at[idx_c.at[0]], ssem[b]).wait()

    _gather(0, 0)
    _gather(1, 1)

    def group(t, carry):
        j0 = 4 * t
        for b in range(4):
            j = j0 + b
            sb = (b + 2) % 4

            @pl.when(j >= 2)
            def _():
                _wait_scatter(sb)

            @pl.when(j + 2 < NCH)
            def _():
                _gather(j + 2, sb)

            _wait_gather(j, b)
            _scatter(j, b)
        return carry

    lax.fori_loop(0, NCH // 4, group, 0)
    _wait_gather(NCH - 1, 0)
    _scatter(NCH - 1, 0)
    for b in (0, 2, 3):
        _wait_scatter(b)

    plsc.subcore_barrier()
    pltpu.sync_copy(acc_sh.at[pl.ds(base, ROWS_PER_TILE)],
                    out_hbm.at[c].at[pl.ds(base, ROWS_PER_TILE)])


_spmm = pl.kernel(
    _spmm_body,
    out_type=jax.ShapeDtypeStruct((NC, N_NODES, HID), jnp.float32),
    mesh=_sc_mesh,
    compiler_params=pltpu.CompilerParams(use_tc_tiling_on_sc=False),
    scratch_types=[
        pltpu.VMEM((NCH, CH), jnp.int32),
        pltpu.VMEM((NCH, CH), jnp.int32),
        pltpu.VMEM((CH, HID), jnp.float32),
        pltpu.VMEM((CH, HID), jnp.float32),
        pltpu.VMEM((CH, HID), jnp.float32),
        pltpu.VMEM((CH, HID), jnp.float32),
        pltpu.VMEM_SHARED((N_NODES, HID), jnp.float32),
        pltpu.SemaphoreType.DMA,
        pltpu.SemaphoreType.DMA,
        pltpu.SemaphoreType.DMA,
        pltpu.SemaphoreType.DMA,
        pltpu.SemaphoreType.DMA,
        pltpu.SemaphoreType.DMA,
        pltpu.SemaphoreType.DMA,
        pltpu.SemaphoreType.DMA,
    ],
)


def _deg_body(col_hbm, out_hbm, idx_c, buf, acc_sh):
    c = lax.axis_index("c")
    s = lax.axis_index("s")
    w = c * NS + s
    pltpu.sync_copy(col_hbm.at[w], idx_c)
    base = s * ROWS_PER_TILE
    _zero_acc_slice(buf, acc_sh, base)

    one16 = jnp.ones((16,), jnp.float32)

    def orow(i, carry):
        buf[i, :] = one16
        return carry

    lax.fori_loop(0, CH, orow, 0)
    plsc.subcore_barrier()

    def chunk(j, carry):
        pltpu.sync_copy(buf, acc_sh.at[idx_c.at[j]], add=True)
        return carry

    lax.fori_loop(0, NCH, chunk, 0)
    plsc.subcore_barrier()
    pltpu.sync_copy(acc_sh.at[pl.ds(base, ROWS_PER_TILE)],
                    out_hbm.at[c].at[pl.ds(base, ROWS_PER_TILE)])


_deg = pl.kernel(
    _deg_body,
    out_type=jax.ShapeDtypeStruct((NC, N_NODES, DEG_W), jnp.float32),
    mesh=_sc_mesh,
    compiler_params=pltpu.CompilerParams(use_tc_tiling_on_sc=False),
    scratch_types=[
        pltpu.VMEM((NCH, CH), jnp.int32),
        pltpu.VMEM((CH, DEG_W), jnp.float32),
        pltpu.VMEM_SHARED((N_NODES, DEG_W), jnp.float32),
    ],
)


BR = 1000
GRID = N_NODES // BR
PGRID = PROWS // BR


def _embed_body(x_ref, we_ref, b_ref, x0_ref):
    x0 = jnp.dot(x_ref[...], we_ref[...], preferred_element_type=jnp.float32)
    x0_ref[...] = jnp.maximum(x0 + b_ref[...], 0.0)


_embed = pl.pallas_call(
    _embed_body,
    grid=(GRID,),
    in_specs=[
        pl.BlockSpec((BR, IN_C), lambda i: (i, 0)),
        pl.BlockSpec((IN_C, HID), lambda i: (0, 0)),
        pl.BlockSpec((1, HID), lambda i: (0, 0)),
    ],
    out_specs=pl.BlockSpec((BR, HID), lambda i: (i, 0)),
    out_shape=jax.ShapeDtypeStruct((N_NODES, HID), jnp.float32),
)


def _scale_body(db_ref, x0_ref, g_ref, v_ref, u_ref):
    dv = lax.rsqrt(db_ref[...] + 1.0)
    x0 = x0_ref[...]
    g_ref[...] = dv * x0
    v_ref[...] = (1.0 - ALPHA) * dv
    u_ref[...] = ALPHA * x0


_scale = pl.pallas_call(
    _scale_body,
    grid=(PGRID,),
    in_specs=[
        pl.BlockSpec((BR, PW), lambda i: (i, 0)),
        pl.BlockSpec((BR, PW), lambda i: (i, 0)),
    ],
    out_specs=[
        pl.BlockSpec((BR, PW), lambda i: (i, 0)),
        pl.BlockSpec((BR, PW), lambda i: (i, 0)),
        pl.BlockSpec((BR, PW), lambda i: (i, 0)),
    ],
    out_shape=[
        jax.ShapeDtypeStruct((PROWS, PW), jnp.float32),
        jax.ShapeDtypeStruct((PROWS, PW), jnp.float32),
        jax.ShapeDtypeStruct((PROWS, PW), jnp.float32),
    ],
)


def _layer_body(acc_ref, v_ref, u_ref, w_ref, o_ref, *, beta, final):
    t = v_ref[...] * (acc_ref[0] + acc_ref[1]) + u_ref[...]
    out = (1.0 - beta) * t + beta * jnp.dot(
        t, w_ref[...], preferred_element_type=jnp.float32)
    if final:
        o_ref[...] = out
    else:
        o_ref[...] = (1.0 / (1.0 - ALPHA)) * v_ref[...] * jnp.maximum(out, 0.0)


def _make_layer(beta, final):
    return pl.pallas_call(
        functools.partial(_layer_body, beta=beta, final=final),
        grid=(PGRID,),
        in_specs=[
            pl.BlockSpec((NC, BR, PW), lambda i: (0, i, 0)),
            pl.BlockSpec((BR, PW), lambda i: (i, 0)),
            pl.BlockSpec((BR, PW), lambda i: (i, 0)),
            pl.BlockSpec((PW, PW), lambda i: (0, 0)),
        ],
        out_specs=pl.BlockSpec((BR, PW), lambda i: (i, 0)),
        out_shape=jax.ShapeDtypeStruct((PROWS, PW), jnp.float32),
    )


_layers = [_make_layer(math.log(THETA / l + 1.0), final=(l == 4))
           for l in range(1, 5)]


def kernel(x, adj_t, W_embed, b_embed, W1, W2, W3, W4):
    adj = adj_t.astype(jnp.int32)
    row3 = adj[0].reshape(NW, NCH, CH)
    col3 = adj[1].reshape(NW, NCH, CH)
    z = jnp.zeros((HID, HID), jnp.float32)
    wpad = [jnp.concatenate([jnp.concatenate([W, z], 1),
                             jnp.concatenate([z, W], 1)], 0)
            for W in (W1, W2, W3, W4)]

    degp = _deg(col3)
    x0 = _embed(x, W_embed, b_embed.reshape(1, HID))
    degb = jnp.broadcast_to(degp[0, :, 0:1] + degp[1, :, 0:1],
                            (N_NODES, HID)).reshape(PROWS, PW)
    g, v, u = _scale(degb, x0.reshape(PROWS, PW))
    for lyr, W2p in zip(_layers, wpad):
        acc = _spmm(g.reshape(N_NODES, HID), row3, col3)
        g = lyr(acc.reshape(NC, PROWS, PW), v, u, W2p)
    return g.reshape(N_NODES, HID)


# speedup vs baseline: 1.4704x; 1.0311x over previous
import functools
import math

import jax
import jax.numpy as jnp
from jax import lax
from jax.experimental import pallas as pl
from jax.experimental.pallas import tpu as pltpu
from jax.experimental.pallas import tpu_sc as plsc

N_NODES = 10000
N_EDGES = 320000
IN_C = 128
HID = 64
ALPHA = 0.1
THETA = 0.5

NC, NS = 2, 16
NW = NC * NS
CH = 80
NCH = N_EDGES // (NW * CH)
ROWS_PER_TILE = N_NODES // NS
DEG_W = 16

PROWS = N_NODES // 2
PW = 2 * HID

_sc_mesh = plsc.VectorSubcoreMesh(core_axis_name="c", subcore_axis_name="s")


def _zero_acc_slice(buf, acc_sh, base):
    z16 = jnp.zeros((16,), jnp.float32)
    width = buf.shape[1]

    def zrow(i, carry):
        for j in range(width // 16):
            buf[i, pl.ds(j * 16, 16)] = z16
        return carry

    lax.fori_loop(0, CH, zrow, 0)
    for k in range(ROWS_PER_TILE // CH):
        pltpu.sync_copy(buf, acc_sh.at[pl.ds(base + k * CH, CH)])
    rem = ROWS_PER_TILE % CH
    if rem:
        pltpu.sync_copy(buf.at[pl.ds(0, rem)],
                        acc_sh.at[pl.ds(base + (ROWS_PER_TILE // CH) * CH, rem)])


def _spmm_body(g_hbm, row_hbm, col_hbm, out_hbm,
               idx_r, idx_c, b0, b1, b2, b3, acc_sh,
               g0, g1, g2, g3, s0, s1, s2, s3):
    bufs = (b0, b1, b2, b3)
    gsem = (g0, g1, g2, g3)
    ssem = (s0, s1, s2, s3)
    c = lax.axis_index("c")
    s = lax.axis_index("s")
    w = c * NS + s
    pltpu.sync_copy(row_hbm.at[w], idx_r)
    pltpu.sync_copy(col_hbm.at[w], idx_c)
    base = s * ROWS_PER_TILE

    @pl.when(c == 0)
    def _():
        pltpu.sync_copy(g_hbm.at[pl.ds(base, ROWS_PER_TILE)],
                        acc_sh.at[pl.ds(base, ROWS_PER_TILE)])

    @pl.when(c == 1)
    def _():
        _zero_acc_slice(bufs[0], acc_sh, base)

    plsc.subcore_barrier()

    def _gather(j, b):
        pltpu.async_copy(g_hbm.at[idx_r.at[j]], bufs[b], gsem[b])

    def _wait_gather(j, b):
        pltpu.make_async_copy(g_hbm.at[idx_r.at[j]], bufs[b], gsem[b]).wait()

    def _scatter(j, b):
        pltpu.async_copy(bufs[b], acc_sh.at[idx_c.at[j]], ssem[b], add=True)

    def _wait_scatter(b):
        pltpu.make_async_copy(bufs[b], acc_sh.at[idx_c.at[0]], ssem[b]).wait()

    _gather(0, 0)
    _gather(1, 1)

    def group(t, carry):
        j0 = 4 * t
        for b in range(4):
            j = j0 + b
            sb = (b + 2) % 4

            @pl.when(j >= 2)
            def _():
                _wait_scatter(sb)

            @pl.when(j + 2 < NCH)
            def _():
                _gather(j + 2, sb)

            _wait_gather(j, b)
            _scatter(j, b)
        return carry

    lax.fori_loop(0, NCH // 4, group, 0)
    _wait_gather(NCH - 1, 0)
    _scatter(NCH - 1, 0)
    for b in (0, 2, 3):
        _wait_scatter(b)

    plsc.subcore_barrier()
    pltpu.sync_copy(acc_sh.at[pl.ds(base, ROWS_PER_TILE)],
                    out_hbm.at[c].at[pl.ds(base, ROWS_PER_TILE)])


_spmm = pl.kernel(
    _spmm_body,
    out_type=jax.ShapeDtypeStruct((NC, N_NODES, HID), jnp.float32),
    mesh=_sc_mesh,
    compiler_params=pltpu.CompilerParams(use_tc_tiling_on_sc=False),
    scratch_types=[
        pltpu.VMEM((NCH, CH), jnp.int32),
        pltpu.VMEM((NCH, CH), jnp.int32),
        pltpu.VMEM((CH, HID), jnp.float32),
        pltpu.VMEM((CH, HID), jnp.float32),
        pltpu.VMEM((CH, HID), jnp.float32),
        pltpu.VMEM((CH, HID), jnp.float32),
        pltpu.VMEM_SHARED((N_NODES, HID), jnp.float32),
        pltpu.SemaphoreType.DMA,
        pltpu.SemaphoreType.DMA,
        pltpu.SemaphoreType.DMA,
        pltpu.SemaphoreType.DMA,
        pltpu.SemaphoreType.DMA,
        pltpu.SemaphoreType.DMA,
        pltpu.SemaphoreType.DMA,
        pltpu.SemaphoreType.DMA,
    ],
)


def _deg_body(col_hbm, out_hbm, idx_c, buf, acc_sh):
    c = lax.axis_index("c")
    s = lax.axis_index("s")
    w = c * NS + s
    pltpu.sync_copy(col_hbm.at[w], idx_c)
    base = s * ROWS_PER_TILE
    _zero_acc_slice(buf, acc_sh, base)

    one16 = jnp.ones((16,), jnp.float32)

    def orow(i, carry):
        buf[i, :] = one16
        return carry

    lax.fori_loop(0, CH, orow, 0)
    plsc.subcore_barrier()

    def chunk(j, carry):
        pltpu.sync_copy(buf, acc_sh.at[idx_c.at[j]], add=True)
        return carry

    lax.fori_loop(0, NCH, chunk, 0)
    plsc.subcore_barrier()
    pltpu.sync_copy(acc_sh.at[pl.ds(base, ROWS_PER_TILE)],
                    out_hbm.at[c].at[pl.ds(base, ROWS_PER_TILE)])


_deg = pl.kernel(
    _deg_body,
    out_type=jax.ShapeDtypeStruct((NC, N_NODES, DEG_W), jnp.float32),
    mesh=_sc_mesh,
    compiler_params=pltpu.CompilerParams(use_tc_tiling_on_sc=False),
    scratch_types=[
        pltpu.VMEM((NCH, CH), jnp.int32),
        pltpu.VMEM((CH, DEG_W), jnp.float32),
        pltpu.VMEM_SHARED((N_NODES, DEG_W), jnp.float32),
    ],
)


BR = 1000
GRID = N_NODES // BR
PGRID = PROWS // BR
ER = N_EDGES // 128


def _delin_body(a_ref, r_ref, c_ref):
    r_ref[...] = jnp.reshape(a_ref[0], (ER, 128))
    c_ref[...] = jnp.reshape(a_ref[1], (ER, 128))


_delin = pl.pallas_call(
    _delin_body,
    in_specs=[pl.BlockSpec((2, N_EDGES), lambda: (0, 0))],
    out_specs=[
        pl.BlockSpec((ER, 128), lambda: (0, 0)),
        pl.BlockSpec((ER, 128), lambda: (0, 0)),
    ],
    out_shape=[
        jax.ShapeDtypeStruct((ER, 128), jnp.int32),
        jax.ShapeDtypeStruct((ER, 128), jnp.int32),
    ],
)


def _embed_body(x_ref, we_ref, b_ref, x0_ref):
    x0 = jnp.dot(x_ref[...], we_ref[...], preferred_element_type=jnp.float32)
    x0_ref[...] = jnp.maximum(x0 + b_ref[...], 0.0)


_embed = pl.pallas_call(
    _embed_body,
    grid=(GRID,),
    in_specs=[
        pl.BlockSpec((BR, IN_C), lambda i: (i, 0)),
        pl.BlockSpec((IN_C, HID), lambda i: (0, 0)),
        pl.BlockSpec((1, HID), lambda i: (0, 0)),
    ],
    out_specs=pl.BlockSpec((BR, HID), lambda i: (i, 0)),
    out_shape=jax.ShapeDtypeStruct((N_NODES, HID), jnp.float32),
)


def _scale_body(db_ref, x0_ref, g_ref, v_ref, u_ref):
    dv = lax.rsqrt(db_ref[...] + 1.0)
    x0 = x0_ref[...]
    g_ref[...] = dv * x0
    v_ref[...] = (1.0 - ALPHA) * dv
    u_ref[...] = ALPHA * x0


_scale = pl.pallas_call(
    _scale_body,
    grid=(PGRID,),
    in_specs=[
        pl.BlockSpec((BR, PW), lambda i: (i, 0)),
        pl.BlockSpec((BR, PW), lambda i: (i, 0)),
    ],
    out_specs=[
        pl.BlockSpec((BR, PW), lambda i: (i, 0)),
        pl.BlockSpec((BR, PW), lambda i: (i, 0)),
        pl.BlockSpec((BR, PW), lambda i: (i, 0)),
    ],
    out_shape=[
        jax.ShapeDtypeStruct((PROWS, PW), jnp.float32),
        jax.ShapeDtypeStruct((PROWS, PW), jnp.float32),
        jax.ShapeDtypeStruct((PROWS, PW), jnp.float32),
    ],
)


def _layer_body(acc_ref, v_ref, u_ref, w_ref, o_ref, *, beta, final):
    t = v_ref[...] * (acc_ref[0] + acc_ref[1]) + u_ref[...]
    out = (1.0 - beta) * t + beta * jnp.dot(
        t, w_ref[...], preferred_element_type=jnp.float32)
    if final:
        o_ref[...] = out
    else:
        o_ref[...] = (1.0 / (1.0 - ALPHA)) * v_ref[...] * jnp.maximum(out, 0.0)


def _make_layer(beta, final):
    out_spec = pl.BlockSpec((BR, PW), lambda i: (i, 0))
    out_shape = jax.ShapeDtypeStruct((PROWS, PW), jnp.float32)
    return pl.pallas_call(
        functools.partial(_layer_body, beta=beta, final=final),
        grid=(PGRID,),
        in_specs=[
            pl.BlockSpec((NC, BR, PW), lambda i: (0, i, 0)),
            pl.BlockSpec((BR, PW), lambda i: (i, 0)),
            pl.BlockSpec((BR, PW), lambda i: (i, 0)),
            pl.BlockSpec((PW, PW), lambda i: (0, 0)),
        ],
        out_specs=out_spec,
        out_shape=out_shape,
    )


_layers = [_make_layer(math.log(THETA / l + 1.0), final=(l == 4))
           for l in range(1, 5)]


def kernel(x, adj_t, W_embed, b_embed, W1, W2, W3, W4):
    row_f, col_f = _delin(adj_t.astype(jnp.int32))
    row3 = row_f.reshape(NW, NCH, CH)
    col3 = col_f.reshape(NW, NCH, CH)
    z = jnp.zeros((HID, HID), jnp.float32)
    wpad = [jnp.concatenate([jnp.concatenate([W, z], 1),
                             jnp.concatenate([z, W], 1)], 0)
            for W in (W1, W2, W3, W4)]

    degp = _deg(col3)
    x0 = _embed(x, W_embed, b_embed.reshape(1, HID))
    degb = jnp.broadcast_to(degp[0, :, 0:1] + degp[1, :, 0:1],
                            (N_NODES, HID)).reshape(PROWS, PW)
    g, v, u = _scale(degb, x0.reshape(PROWS, PW))
    for lyr, W2p in zip(_layers, wpad):
        acc = _spmm(g.reshape(N_NODES, HID), row3, col3)
        g = lyr(acc.reshape(NC, PROWS, PW), v, u, W2p)
    return g.reshape(N_NODES, HID)
